# Initial kernel scaffold; baseline (speedup 1.0000x reference)
import jax
import jax.numpy as jnp
from jax.experimental import pallas as pl


def kernel(conv1_w, conv1_b, conv2_w, conv2_b, conv3_w, conv3_b, fc_w, fc_b, emb_table, rtg_w, rtg_b, w_ih, w_hh, b_ih, b_hh, head_w, frames, reward_to_go, previous_actions):
    raise NotImplementedError("write your pallas kernel here")


# speedup vs baseline: 10.2057x; 10.2057x over previous
import functools

import numpy as np
import jax
import jax.numpy as jnp
from jax.experimental import pallas as pl
from jax.experimental.pallas import tpu as pltpu

_E = 256
_HD = 128
_NA = 41


def _tower_kernel(x_ref, w1_ref, b1_ref, w2_ref, b2_ref, w3_ref, b3_ref,
                  o_ref):
    nb = x_ref.shape[0]
    m = 4 * nb

    def act(v, w, bias):
        return jnp.maximum(
            jnp.dot(v, w, preferred_element_type=jnp.float32) + bias, 0.0)

    xr = []
    for r1 in range(3):
        pieces = []
        for r3 in range(4):
            for r2 in range(3):
                for oh3 in range(4):
                    h = 36 * oh3 + 9 * r3 + 3 * r2 + r1
                    pieces.append(x_ref[:, 0, h, :])
        xr.append(jnp.concatenate(pieces, axis=0))

    y1p = []
    for g in range(6):
        p = jnp.concatenate([xr[r1][:, g * 72:(g + 1) * 72]
                             for r1 in range(3)], axis=1)
        y1p.append(act(p, w1_ref[...], b1_ref[...]))
    y1 = jnp.concatenate(y1p, axis=1)

    y2p = []
    for g2 in range(4):
        p = jnp.concatenate(
            [jnp.concatenate(
                [y1[(r3 * 3 + r2) * m:(r3 * 3 + r2 + 1) * m,
                    g2 * 192:(g2 + 1) * 192] for r2 in range(3)], axis=1)
             for r3 in range(4)], axis=0)
        y2p.append(act(p, w2_ref[...], b2_ref[...]))
    y2 = jnp.concatenate(y2p, axis=1)

    p3 = jnp.concatenate([y2[r3 * m:(r3 + 1) * m, :]
                          for r3 in range(4)], axis=1)
    y3 = act(p3, w3_ref[...], b3_ref[...])

    for oh3 in range(4):
        o_ref[0, :, oh3 * 256:(oh3 + 1) * 256] = \
            y3[oh3 * nb:(oh3 + 1) * nb, :]


def _conv_tower(frames4, w1g, b1g, w2m, b2, w3m, b3, *, b, f, nb):
    jb = b // nb
    return pl.pallas_call(
        _tower_kernel,
        out_shape=jax.ShapeDtypeStruct((f * jb, nb, 1024), jnp.float32),
        grid_spec=pltpu.PrefetchScalarGridSpec(
            num_scalar_prefetch=0,
            grid=(jb, f),
            in_specs=[
                pl.BlockSpec((nb, 1, 144, 432), lambda j, t: (j, t, 0, 0)),
                pl.BlockSpec((216, 128), lambda j, t: (0, 0)),
                pl.BlockSpec((1, 128), lambda j, t: (0, 0)),
                pl.BlockSpec((576, 128), lambda j, t: (0, 0)),
                pl.BlockSpec((1, 128), lambda j, t: (0, 0)),
                pl.BlockSpec((2048, 256), lambda j, t: (0, 0)),
                pl.BlockSpec((1, 256), lambda j, t: (0, 0)),
            ],
            out_specs=pl.BlockSpec((1, nb, 1024),
                                   lambda j, t: (t * jb + j, 0, 0)),
        ),
        compiler_params=pltpu.CompilerParams(
            dimension_semantics=("parallel", "parallel")),
    )(frames4, w1g, b1g, w2m, b2, w3m, b3)


def _tail_kernel(feat_ref, fcw_ref, fcb_ref, act_ref, emb_ref, rtg_ref,
                 rtgw_ref, rtgb_ref, wih_ref, whh_ref, bg_ref, head_ref,
                 logits_ref, hn_ref, cn_ref, h_scr, *, batch, n_steps,
                 hidden, n_actions):
    tb = batch * n_steps
    hd = hidden

    vis = jnp.tanh(
        jnp.dot(feat_ref[...], fcw_ref[...],
                preferred_element_type=jnp.float32) + fcb_ref[...])
    onehot = (jax.lax.broadcasted_iota(jnp.int32, (tb, n_actions), 1)
              == act_ref[...]).astype(jnp.float32)
    aemb = jnp.tanh(jnp.dot(onehot, emb_ref[...],
                            preferred_element_type=jnp.float32))
    remb = jnp.tanh(rtg_ref[...] * rtgw_ref[...] + rtgb_ref[...])
    zin = vis + aemb + remb

    gx = (jnp.dot(zin, wih_ref[...], preferred_element_type=jnp.float32)
          + bg_ref[...])
    whh = whh_ref[...]
    h = jnp.zeros((batch, hd), jnp.float32)
    c = jnp.zeros((batch, hd), jnp.float32)
    for t in range(n_steps):
        g = gx[t * batch:(t + 1) * batch, :] + jnp.dot(
            h, whh, preferred_element_type=jnp.float32)
        s = jax.nn.sigmoid(g[:, :3 * hd])
        gg = jnp.tanh(g[:, 3 * hd:])
        c = s[:, hd:2 * hd] * c + s[:, :hd] * gg
        h = s[:, 2 * hd:3 * hd] * jnp.tanh(c)
        h_scr[t * batch:(t + 1) * batch, :] = h
    logits_ref[...] = jnp.dot(h_scr[...], head_ref[...],
                              preferred_element_type=jnp.float32)
    hn_ref[...] = h
    cn_ref[...] = c


def _tail(feat_tm, fcw_t, fc_b, acts_tm, emb_table, rtg_tm, rtg_w, rtg_b,
          wih_t, whh_t, bg, head_pad, *, batch, n_steps, hidden, n_actions):
    tb = batch * n_steps
    e = fcw_t.shape[1]
    nh = head_pad.shape[1]

    def full(shape):
        return pl.BlockSpec(shape, lambda i: (0,) * len(shape))

    body = functools.partial(_tail_kernel, batch=batch, n_steps=n_steps,
                             hidden=hidden, n_actions=n_actions)
    return pl.pallas_call(
        body,
        out_shape=(
            jax.ShapeDtypeStruct((tb, nh), jnp.float32),
            jax.ShapeDtypeStruct((batch, hidden), jnp.float32),
            jax.ShapeDtypeStruct((batch, hidden), jnp.float32),
        ),
        grid_spec=pltpu.PrefetchScalarGridSpec(
            num_scalar_prefetch=0,
            grid=(1,),
            in_specs=[
                full((tb, 1024)), full((1024, e)), full((1, e)),
                full((tb, 1)), full(emb_table.shape), full((tb, 1)),
                full((1, e)), full((1, e)), full((e, 4 * hidden)),
                full((hidden, 4 * hidden)), full((1, 4 * hidden)),
                full((hidden, nh)),
            ],
            out_specs=(full((tb, nh)), full((batch, hidden)),
                       full((batch, hidden))),
            scratch_shapes=[pltpu.VMEM((tb, hidden), jnp.float32)],
        ),
        compiler_params=pltpu.CompilerParams(
            dimension_semantics=("arbitrary",)),
    )(feat_tm, fcw_t, fc_b.reshape(1, e), acts_tm, emb_table, rtg_tm,
      rtg_w.reshape(1, e), rtg_b.reshape(1, e), wih_t, whh_t,
      bg.reshape(1, 4 * hidden), head_pad)


def kernel(conv1_w, conv1_b, conv2_w, conv2_b, conv3_w, conv3_b, fc_w, fc_b,
           emb_table, rtg_w, rtg_b, w_ih, w_hh, b_ih, b_hh, head_w,
           frames, reward_to_go, previous_actions):
    b, f, img, _, _ = frames.shape
    e, hd, a = _E, _HD, _NA
    nb = 8 if b % 8 == 0 else (2 if b % 2 == 0 else 1)

    frames4 = frames.astype(jnp.float32).reshape(b, f, 144, 432)

    eye8 = jnp.eye(8, dtype=jnp.float32)
    eye4 = jnp.eye(4, dtype=jnp.float32)
    w1r = jnp.transpose(conv1_w.astype(jnp.float32),
                        (2, 3, 1, 0)).reshape(3, 9, 16)
    w1g = (eye8[None, :, None, :, None]
           * w1r[:, None, :, None, :]).reshape(216, 128)
    b1g = jnp.tile(conv1_b.astype(jnp.float32).reshape(1, 16), (1, 8))

    w2r = jnp.transpose(conv2_w.astype(jnp.float32),
                        (2, 3, 1, 0)).reshape(3, 48, 32)
    w2g = (eye4[None, :, None, :, None]
           * w2r[:, None, :, None, :]).reshape(576, 128)
    b2g = jnp.tile(conv2_b.astype(jnp.float32).reshape(1, 32), (1, 4))

    w3r = jnp.transpose(conv3_w.astype(jnp.float32),
                        (2, 3, 1, 0)).reshape(4, 128, 64)
    w3g = (eye4[None, :, None, :, None]
           * w3r[:, None, :, None, :]).reshape(2048, 256)
    b3g = jnp.tile(conv3_b.astype(jnp.float32).reshape(1, 64), (1, 4))

    feat_tm = _conv_tower(frames4, w1g, b1g, w2g, b2g, w3g, b3g,
                          b=b, f=f, nb=nb).reshape(f * b, 1024)

    hh = np.arange(4).reshape(4, 1, 1)
    ww = np.arange(4).reshape(1, 4, 1)
    cc = np.arange(64).reshape(1, 1, 64)
    perm = jnp.asarray((cc * 16 + hh * 4 + ww).reshape(-1))
    fcw_t = fc_w.astype(jnp.float32)[:, perm].T

    gperm = jnp.asarray(np.concatenate(
        [np.arange(0, 2 * hd), np.arange(3 * hd, 4 * hd),
         np.arange(2 * hd, 3 * hd)]))
    wih_t = w_ih.astype(jnp.float32)[gperm].T
    whh_t = w_hh.astype(jnp.float32)[gperm].T
    bg = (b_ih + b_hh).astype(jnp.float32)[gperm]

    nh = -(-a // 128) * 128
    head_pad = jnp.pad(head_w.astype(jnp.float32).T, ((0, 0), (0, nh - a)))

    tb = b * f
    acts_tm = jnp.swapaxes(previous_actions, 0, 1).reshape(tb, 1)
    rtg_tm = jnp.swapaxes(reward_to_go.astype(jnp.float32),
                          0, 1).reshape(tb, 1)

    logits_pad, hn, cn = _tail(
        feat_tm, fcw_t, fc_b.astype(jnp.float32), acts_tm.astype(jnp.int32),
        emb_table.astype(jnp.float32), rtg_tm, rtg_w.astype(jnp.float32).T,
        rtg_b.astype(jnp.float32), wih_t, whh_t, bg, head_pad,
        batch=b, n_steps=f, hidden=hd, n_actions=a)

    logits = jnp.transpose(logits_pad.reshape(f, b, nh)[:, :, :a], (1, 0, 2))
    return logits, (hn[None], cn[None])


# speedup vs baseline: 17.3483x; 1.6999x over previous
import functools

import numpy as np
import jax
import jax.numpy as jnp
from jax.experimental import pallas as pl
from jax.experimental.pallas import tpu as pltpu

_E = 256
_HD = 128
_NA = 41


def _tower_kernel(x_ref, w1_ref, b1_ref, w2_ref, b2_ref, w3_ref, b3_ref,
                  o_ref):
    nb = x_ref.shape[0]
    m = 4 * nb

    def act(v, w, bias):
        return jnp.maximum(
            jnp.dot(v, w, preferred_element_type=jnp.float32) + bias, 0.0)

    acc = b1_ref[...]
    for r1 in range(3):
        for c in range(3):
            pieces = []
            for r3 in range(4):
                for r2 in range(3):
                    for oh3 in range(4):
                        h = 36 * oh3 + 9 * r3 + 3 * r2 + r1
                        pieces.append(x_ref[:, 0, c, h, :])
            xx = jnp.concatenate(pieces, axis=0)
            acc = acc + jnp.dot(xx, w1_ref[r1, c],
                                preferred_element_type=jnp.float32)
    y1 = jnp.maximum(acc, 0.0)

    y2p = []
    for g2 in range(4):
        p = jnp.concatenate(
            [jnp.concatenate(
                [y1[(r3 * 3 + r2) * m:(r3 * 3 + r2 + 1) * m,
                    g2 * 192:(g2 + 1) * 192] for r2 in range(3)], axis=1)
             for r3 in range(4)], axis=0)
        y2p.append(act(p, w2_ref[...], b2_ref[...]))
    y2 = jnp.concatenate(y2p, axis=1)

    p3 = jnp.concatenate([y2[r3 * m:(r3 + 1) * m, :]
                          for r3 in range(4)], axis=1)
    y3 = act(p3, w3_ref[...], b3_ref[...])

    for oh3 in range(4):
        o_ref[0, :, oh3 * 256:(oh3 + 1) * 256] = \
            y3[oh3 * nb:(oh3 + 1) * nb, :]


def _conv_tower(frames4, w1g, b1g, w2m, b2, w3m, b3, *, b, f, nb):
    jb = b // nb
    return pl.pallas_call(
        _tower_kernel,
        out_shape=jax.ShapeDtypeStruct((f * jb, nb, 1024), jnp.float32),
        grid_spec=pltpu.PrefetchScalarGridSpec(
            num_scalar_prefetch=0,
            grid=(jb, f),
            in_specs=[
                pl.BlockSpec((nb, 1, 3, 144, 144),
                             lambda j, t: (j, t, 0, 0, 0)),
                pl.BlockSpec((3, 3, 144, 768), lambda j, t: (0, 0, 0, 0)),
                pl.BlockSpec((1, 768), lambda j, t: (0, 0)),
                pl.BlockSpec((576, 128), lambda j, t: (0, 0)),
                pl.BlockSpec((1, 128), lambda j, t: (0, 0)),
                pl.BlockSpec((2048, 256), lambda j, t: (0, 0)),
                pl.BlockSpec((1, 256), lambda j, t: (0, 0)),
            ],
            out_specs=pl.BlockSpec((1, nb, 1024),
                                   lambda j, t: (t * jb + j, 0, 0)),
        ),
        compiler_params=pltpu.CompilerParams(
            dimension_semantics=("parallel", "parallel")),
    )(frames4, w1g, b1g, w2m, b2, w3m, b3)


def _tail_kernel(feat_ref, fcw_ref, fcb_ref, act_ref, emb_ref, rtg_ref,
                 rtgw_ref, rtgb_ref, wih_ref, whh_ref, bg_ref, head_ref,
                 logits_ref, hn_ref, cn_ref, h_scr, *, batch, n_steps,
                 hidden, n_actions):
    tb = batch * n_steps
    hd = hidden

    vis = jnp.tanh(
        jnp.dot(feat_ref[...], fcw_ref[...],
                preferred_element_type=jnp.float32) + fcb_ref[...])
    onehot = (jax.lax.broadcasted_iota(jnp.int32, (tb, n_actions), 1)
              == act_ref[...]).astype(jnp.float32)
    aemb = jnp.tanh(jnp.dot(onehot, emb_ref[...],
                            preferred_element_type=jnp.float32))
    remb = jnp.tanh(rtg_ref[...] * rtgw_ref[...] + rtgb_ref[...])
    zin = vis + aemb + remb

    gx = (jnp.dot(zin, wih_ref[...], preferred_element_type=jnp.float32)
          + bg_ref[...])
    whh = whh_ref[...]
    h = jnp.zeros((batch, hd), jnp.float32)
    c = jnp.zeros((batch, hd), jnp.float32)
    for t in range(n_steps):
        g = gx[t * batch:(t + 1) * batch, :] + jnp.dot(
            h, whh, preferred_element_type=jnp.float32)
        s = jax.nn.sigmoid(g[:, :3 * hd])
        gg = jnp.tanh(g[:, 3 * hd:])
        c = s[:, hd:2 * hd] * c + s[:, :hd] * gg
        h = s[:, 2 * hd:3 * hd] * jnp.tanh(c)
        h_scr[t * batch:(t + 1) * batch, :] = h
    logits_ref[...] = jnp.dot(h_scr[...], head_ref[...],
                              preferred_element_type=jnp.float32)
    hn_ref[...] = h
    cn_ref[...] = c


def _tail(feat_tm, fcw_t, fc_b, acts_tm, emb_table, rtg_tm, rtg_w, rtg_b,
          wih_t, whh_t, bg, head_pad, *, batch, n_steps, hidden, n_actions):
    tb = batch * n_steps
    e = fcw_t.shape[1]
    nh = head_pad.shape[1]

    def full(shape):
        return pl.BlockSpec(shape, lambda i: (0,) * len(shape))

    body = functools.partial(_tail_kernel, batch=batch, n_steps=n_steps,
                             hidden=hidden, n_actions=n_actions)
    return pl.pallas_call(
        body,
        out_shape=(
            jax.ShapeDtypeStruct((tb, nh), jnp.float32),
            jax.ShapeDtypeStruct((batch, hidden), jnp.float32),
            jax.ShapeDtypeStruct((batch, hidden), jnp.float32),
        ),
        grid_spec=pltpu.PrefetchScalarGridSpec(
            num_scalar_prefetch=0,
            grid=(1,),
            in_specs=[
                full((tb, 1024)), full((1024, e)), full((1, e)),
                full((tb, 1)), full(emb_table.shape), full((tb, 1)),
                full((1, e)), full((1, e)), full((e, 4 * hidden)),
                full((hidden, 4 * hidden)), full((1, 4 * hidden)),
                full((hidden, nh)),
            ],
            out_specs=(full((tb, nh)), full((batch, hidden)),
                       full((batch, hidden))),
            scratch_shapes=[pltpu.VMEM((tb, hidden), jnp.float32)],
        ),
        compiler_params=pltpu.CompilerParams(
            dimension_semantics=("arbitrary",)),
    )(feat_tm, fcw_t, fc_b.reshape(1, e), acts_tm, emb_table, rtg_tm,
      rtg_w.reshape(1, e), rtg_b.reshape(1, e), wih_t, whh_t,
      bg.reshape(1, 4 * hidden), head_pad)


def kernel(conv1_w, conv1_b, conv2_w, conv2_b, conv3_w, conv3_b, fc_w, fc_b,
           emb_table, rtg_w, rtg_b, w_ih, w_hh, b_ih, b_hh, head_w,
           frames, reward_to_go, previous_actions):
    b, f, img, _, _ = frames.shape
    e, hd, a = _E, _HD, _NA
    nb = 8 if b % 8 == 0 else (2 if b % 2 == 0 else 1)

    frames5 = jnp.transpose(frames.astype(jnp.float32), (0, 1, 4, 2, 3))

    eye48 = jnp.eye(48, dtype=jnp.float32)
    eye4 = jnp.eye(4, dtype=jnp.float32)
    w1c = jnp.transpose(conv1_w.astype(jnp.float32),
                        (2, 1, 3, 0))
    w1b = (eye48[None, None, :, None, :, None]
           * w1c[:, :, None, :, None, :]).reshape(3, 3, 144, 768)
    b1g = jnp.tile(conv1_b.astype(jnp.float32).reshape(1, 16), (1, 48))

    w2r = jnp.transpose(conv2_w.astype(jnp.float32),
                        (2, 3, 1, 0)).reshape(3, 48, 32)
    w2g = (eye4[None, :, None, :, None]
           * w2r[:, None, :, None, :]).reshape(576, 128)
    b2g = jnp.tile(conv2_b.astype(jnp.float32).reshape(1, 32), (1, 4))

    w3r = jnp.transpose(conv3_w.astype(jnp.float32),
                        (2, 3, 1, 0)).reshape(4, 128, 64)
    w3g = (eye4[None, :, None, :, None]
           * w3r[:, None, :, None, :]).reshape(2048, 256)
    b3g = jnp.tile(conv3_b.astype(jnp.float32).reshape(1, 64), (1, 4))

    feat_tm = _conv_tower(frames5, w1b, b1g, w2g, b2g, w3g, b3g,
                          b=b, f=f, nb=nb).reshape(f * b, 1024)

    hh = np.arange(4).reshape(4, 1, 1)
    ww = np.arange(4).reshape(1, 4, 1)
    cc = np.arange(64).reshape(1, 1, 64)
    perm = jnp.asarray((cc * 16 + hh * 4 + ww).reshape(-1))
    fcw_t = fc_w.astype(jnp.float32)[:, perm].T

    gperm = jnp.asarray(np.concatenate(
        [np.arange(0, 2 * hd), np.arange(3 * hd, 4 * hd),
         np.arange(2 * hd, 3 * hd)]))
    wih_t = w_ih.astype(jnp.float32)[gperm].T
    whh_t = w_hh.astype(jnp.float32)[gperm].T
    bg = (b_ih + b_hh).astype(jnp.float32)[gperm]

    nh = -(-a // 128) * 128
    head_pad = jnp.pad(head_w.astype(jnp.float32).T, ((0, 0), (0, nh - a)))

    tb = b * f
    acts_tm = jnp.swapaxes(previous_actions, 0, 1).reshape(tb, 1)
    rtg_tm = jnp.swapaxes(reward_to_go.astype(jnp.float32),
                          0, 1).reshape(tb, 1)

    logits_pad, hn, cn = _tail(
        feat_tm, fcw_t, fc_b.astype(jnp.float32), acts_tm.astype(jnp.int32),
        emb_table.astype(jnp.float32), rtg_tm, rtg_w.astype(jnp.float32).T,
        rtg_b.astype(jnp.float32), wih_t, whh_t, bg, head_pad,
        batch=b, n_steps=f, hidden=hd, n_actions=a)

    logits = jnp.transpose(logits_pad.reshape(f, b, nh)[:, :, :a], (1, 0, 2))
    return logits, (hn[None], cn[None])


# speedup vs baseline: 18.8535x; 1.0868x over previous
import functools

import numpy as np
import jax
import jax.numpy as jnp
from jax.experimental import pallas as pl
from jax.experimental.pallas import tpu as pltpu

_E = 256
_HD = 128
_NA = 41


def _tower_kernel(x_ref, w1_ref, b1_ref, w2_ref, b2_ref, w3_ref, b3_ref,
                  o_ref):
    nb = x_ref.shape[0]
    m = 4 * nb

    def act(v, w, bias):
        return jnp.maximum(
            jnp.dot(v, w, preferred_element_type=jnp.float32) + bias, 0.0)

    acc = b1_ref[...]
    for r1 in range(3):
        for c in range(3):
            pieces = []
            for r3 in range(4):
                for r2 in range(3):
                    for oh3 in range(4):
                        h = 36 * oh3 + 9 * r3 + 3 * r2 + r1
                        pieces.append(x_ref[:, 0, c, h, :])
            xx = jnp.concatenate(pieces, axis=0)
            acc = acc + jnp.dot(xx, w1_ref[r1, c],
                                preferred_element_type=jnp.float32)
    y1 = jnp.maximum(acc, 0.0)

    y2p = []
    for g2 in range(4):
        p = jnp.concatenate(
            [jnp.concatenate(
                [y1[(r3 * 3 + r2) * m:(r3 * 3 + r2 + 1) * m,
                    g2 * 192:(g2 + 1) * 192] for r2 in range(3)], axis=1)
             for r3 in range(4)], axis=0)
        y2p.append(act(p, w2_ref[...], b2_ref[...]))
    y2 = jnp.concatenate(y2p, axis=1)

    p3 = jnp.concatenate([y2[r3 * m:(r3 + 1) * m, :]
                          for r3 in range(4)], axis=1)
    y3 = act(p3, w3_ref[...], b3_ref[...])

    for oh3 in range(4):
        o_ref[0, :, oh3 * 256:(oh3 + 1) * 256] = \
            y3[oh3 * nb:(oh3 + 1) * nb, :]


def _conv_tower(frames4, w1g, b1g, w2m, b2, w3m, b3, *, b, f, nb):
    jb = b // nb
    return pl.pallas_call(
        _tower_kernel,
        out_shape=jax.ShapeDtypeStruct((f * jb, nb, 1024), jnp.float32),
        grid_spec=pltpu.PrefetchScalarGridSpec(
            num_scalar_prefetch=0,
            grid=(jb, f),
            in_specs=[
                pl.BlockSpec((nb, 1, 3, 144, 144),
                             lambda j, t: (j, t, 0, 0, 0)),
                pl.BlockSpec((3, 3, 144, 768), lambda j, t: (0, 0, 0, 0)),
                pl.BlockSpec((1, 768), lambda j, t: (0, 0)),
                pl.BlockSpec((576, 128), lambda j, t: (0, 0)),
                pl.BlockSpec((1, 128), lambda j, t: (0, 0)),
                pl.BlockSpec((2048, 256), lambda j, t: (0, 0)),
                pl.BlockSpec((1, 256), lambda j, t: (0, 0)),
            ],
            out_specs=pl.BlockSpec((1, nb, 1024),
                                   lambda j, t: (t * jb + j, 0, 0)),
        ),
        compiler_params=pltpu.CompilerParams(
            dimension_semantics=("parallel", "parallel")),
    )(frames4, w1g, b1g, w2m, b2, w3m, b3)


def _tail_kernel(feat_ref, fcw_ref, fcb_ref, act_ref, emb_ref, rtg_ref,
                 rtgw_ref, rtgb_ref, wih_ref, whh_ref, bg_ref, head_ref,
                 logits_ref, hn_ref, cn_ref, h_scr, *, batch, n_steps,
                 hidden, n_actions):
    tb = batch * n_steps
    hd = hidden

    vis = jnp.tanh(
        jnp.dot(feat_ref[...], fcw_ref[...],
                preferred_element_type=jnp.float32) + fcb_ref[...])
    onehot = (jax.lax.broadcasted_iota(jnp.int32, (tb, n_actions), 1)
              == act_ref[...]).astype(jnp.float32)
    aemb = jnp.tanh(jnp.dot(onehot, emb_ref[...],
                            preferred_element_type=jnp.float32))
    remb = jnp.tanh(rtg_ref[...] * rtgw_ref[...] + rtgb_ref[...])
    zin = vis + aemb + remb

    gx = (jnp.dot(zin, wih_ref[...], preferred_element_type=jnp.float32)
          + bg_ref[...])
    whh = whh_ref[...]
    h = jnp.zeros((batch, hd), jnp.float32)
    c = jnp.zeros((batch, hd), jnp.float32)
    for t in range(n_steps):
        g = gx[t * batch:(t + 1) * batch, :] + jnp.dot(
            h, whh, preferred_element_type=jnp.float32)
        s = jax.nn.sigmoid(g[:, :3 * hd])
        gg = jnp.tanh(g[:, 3 * hd:])
        c = s[:, hd:2 * hd] * c + s[:, :hd] * gg
        h = s[:, 2 * hd:3 * hd] * jnp.tanh(c)
        h_scr[t * batch:(t + 1) * batch, :] = h
    logits_ref[...] = jnp.dot(h_scr[...], head_ref[...],
                              preferred_element_type=jnp.float32)
    hn_ref[...] = h
    cn_ref[...] = c


def _tail(feat_tm, fcw_t, fc_b, acts_tm, emb_table, rtg_tm, rtg_w, rtg_b,
          wih_t, whh_t, bg, head_pad, *, batch, n_steps, hidden, n_actions):
    tb = batch * n_steps
    e = fcw_t.shape[1]
    nh = head_pad.shape[1]

    def full(shape):
        return pl.BlockSpec(shape, lambda i: (0,) * len(shape))

    body = functools.partial(_tail_kernel, batch=batch, n_steps=n_steps,
                             hidden=hidden, n_actions=n_actions)
    return pl.pallas_call(
        body,
        out_shape=(
            jax.ShapeDtypeStruct((tb, nh), jnp.float32),
            jax.ShapeDtypeStruct((batch, hidden), jnp.float32),
            jax.ShapeDtypeStruct((batch, hidden), jnp.float32),
        ),
        grid_spec=pltpu.PrefetchScalarGridSpec(
            num_scalar_prefetch=0,
            grid=(1,),
            in_specs=[
                full((tb, 1024)), full((1024, e)), full((1, e)),
                full((tb, 1)), full(emb_table.shape), full((tb, 1)),
                full((1, e)), full((1, e)), full((e, 4 * hidden)),
                full((hidden, 4 * hidden)), full((1, 4 * hidden)),
                full((hidden, nh)),
            ],
            out_specs=(full((tb, nh)), full((batch, hidden)),
                       full((batch, hidden))),
            scratch_shapes=[pltpu.VMEM((tb, hidden), jnp.float32)],
        ),
        compiler_params=pltpu.CompilerParams(
            dimension_semantics=("arbitrary",)),
    )(feat_tm, fcw_t, fc_b.reshape(1, e), acts_tm, emb_table, rtg_tm,
      rtg_w.reshape(1, e), rtg_b.reshape(1, e), wih_t, whh_t,
      bg.reshape(1, 4 * hidden), head_pad)


def kernel(conv1_w, conv1_b, conv2_w, conv2_b, conv3_w, conv3_b, fc_w, fc_b,
           emb_table, rtg_w, rtg_b, w_ih, w_hh, b_ih, b_hh, head_w,
           frames, reward_to_go, previous_actions):
    b, f, img, _, _ = frames.shape
    e, hd, a = _E, _HD, _NA
    nb = 8 if b % 8 == 0 else (2 if b % 2 == 0 else 1)

    frames5 = jnp.transpose(frames.astype(jnp.float32), (0, 1, 4, 2, 3))

    eye4 = np.eye(4, dtype=np.float32)
    eye48 = np.eye(48, dtype=np.float32)
    m1 = jnp.asarray(np.repeat(np.repeat(eye48, 3, axis=0), 16, axis=1))
    w1c = jnp.transpose(conv1_w.astype(jnp.float32),
                        (2, 1, 3, 0))
    w1b = jnp.tile(w1c.reshape(3, 3, 3, 16), (1, 1, 48, 48)) * m1
    b1g = jnp.tile(conv1_b.astype(jnp.float32).reshape(1, 16), (1, 48))

    m2 = jnp.asarray(np.tile(
        np.repeat(np.repeat(eye4, 48, axis=0), 32, axis=1), (3, 1)))
    w2r = jnp.transpose(conv2_w.astype(jnp.float32),
                        (2, 3, 1, 0)).reshape(3, 48, 32)
    w2g = (jnp.tile(w2r, (1, 4, 4)).reshape(576, 128)) * m2
    b2g = jnp.tile(conv2_b.astype(jnp.float32).reshape(1, 32), (1, 4))

    m3 = jnp.asarray(np.tile(
        np.repeat(np.repeat(eye4, 128, axis=0), 64, axis=1), (4, 1)))
    w3r = jnp.transpose(conv3_w.astype(jnp.float32),
                        (2, 3, 1, 0)).reshape(4, 128, 64)
    w3g = (jnp.tile(w3r, (1, 4, 4)).reshape(2048, 256)) * m3
    b3g = jnp.tile(conv3_b.astype(jnp.float32).reshape(1, 64), (1, 4))

    feat_tm = _conv_tower(frames5, w1b, b1g, w2g, b2g, w3g, b3g,
                          b=b, f=f, nb=nb).reshape(f * b, 1024)

    fcw_t = jnp.transpose(fc_w.astype(jnp.float32).reshape(e, 64, 16),
                          (2, 1, 0)).reshape(1024, e)

    def gp(w):
        w = w.astype(jnp.float32)
        return jnp.concatenate(
            [w[:2 * hd], w[3 * hd:], w[2 * hd:3 * hd]], axis=0)

    wih_t = gp(w_ih).T
    whh_t = gp(w_hh).T
    bg = gp((b_ih + b_hh).astype(jnp.float32).reshape(4 * hd, 1)).reshape(-1)

    nh = -(-a // 128) * 128
    head_pad = jnp.pad(head_w.astype(jnp.float32).T, ((0, 0), (0, nh - a)))

    tb = b * f
    acts_tm = jnp.swapaxes(previous_actions, 0, 1).reshape(tb, 1)
    rtg_tm = jnp.swapaxes(reward_to_go.astype(jnp.float32),
                          0, 1).reshape(tb, 1)

    logits_pad, hn, cn = _tail(
        feat_tm, fcw_t, fc_b.astype(jnp.float32), acts_tm.astype(jnp.int32),
        emb_table.astype(jnp.float32), rtg_tm, rtg_w.astype(jnp.float32).T,
        rtg_b.astype(jnp.float32), wih_t, whh_t, bg, head_pad,
        batch=b, n_steps=f, hidden=hd, n_actions=a)

    logits = jnp.transpose(logits_pad.reshape(f, b, nh)[:, :, :a], (1, 0, 2))
    return logits, (hn[None], cn[None])


# speedup vs baseline: 23.2965x; 1.2357x over previous
import functools

import numpy as np
import jax
import jax.numpy as jnp
from jax.experimental import pallas as pl
from jax.experimental.pallas import tpu as pltpu

_E = 256
_HD = 128
_NA = 41


def _tower_kernel(x_ref, w1_ref, b1_ref, w2_ref, b2_ref, w3_ref, b3_ref,
                  o_ref):
    nb = x_ref.shape[0]
    m = 4 * nb

    def act(v, w, bias):
        return jnp.maximum(
            jnp.dot(v, w, preferred_element_type=jnp.float32) + bias, 0.0)

    acc = b1_ref[...]
    for r1 in range(3):
        for c in range(3):
            pieces = []
            for r3 in range(4):
                for r2 in range(3):
                    for oh3 in range(4):
                        h = 36 * oh3 + 9 * r3 + 3 * r2 + r1
                        pieces.append(x_ref[:, 0, c, h, :])
            xx = jnp.concatenate(pieces, axis=0)
            acc = acc + jnp.dot(xx, w1_ref[r1, c],
                                preferred_element_type=jnp.float32)
    y1 = jnp.maximum(acc, 0.0)

    y2p = []
    for g2 in range(4):
        p = jnp.concatenate(
            [jnp.concatenate(
                [y1[(r3 * 3 + r2) * m:(r3 * 3 + r2 + 1) * m,
                    g2 * 192:(g2 + 1) * 192] for r2 in range(3)], axis=1)
             for r3 in range(4)], axis=0)
        y2p.append(act(p, w2_ref[...], b2_ref[...]))
    y2 = jnp.concatenate(y2p, axis=1)

    p3 = jnp.concatenate([y2[r3 * m:(r3 + 1) * m, :]
                          for r3 in range(4)], axis=1)
    y3 = act(p3, w3_ref[...], b3_ref[...])

    for oh3 in range(4):
        o_ref[0, :, oh3 * 256:(oh3 + 1) * 256] = \
            y3[oh3 * nb:(oh3 + 1) * nb, :]


def _conv_tower(frames4, w1g, b1g, w2m, b2, w3m, b3, *, b, f, nb):
    jb = b // nb
    return pl.pallas_call(
        _tower_kernel,
        out_shape=jax.ShapeDtypeStruct((f * jb, nb, 1024), jnp.float32),
        grid_spec=pltpu.PrefetchScalarGridSpec(
            num_scalar_prefetch=0,
            grid=(jb, f),
            in_specs=[
                pl.BlockSpec((nb, 1, 3, 144, 144),
                             lambda j, t: (j, t, 0, 0, 0)),
                pl.BlockSpec((3, 3, 144, 768), lambda j, t: (0, 0, 0, 0)),
                pl.BlockSpec((1, 768), lambda j, t: (0, 0)),
                pl.BlockSpec((576, 128), lambda j, t: (0, 0)),
                pl.BlockSpec((1, 128), lambda j, t: (0, 0)),
                pl.BlockSpec((2048, 256), lambda j, t: (0, 0)),
                pl.BlockSpec((1, 256), lambda j, t: (0, 0)),
            ],
            out_specs=pl.BlockSpec((1, nb, 1024),
                                   lambda j, t: (t * jb + j, 0, 0)),
        ),
        compiler_params=pltpu.CompilerParams(
            dimension_semantics=("parallel", "parallel")),
    )(frames4, w1g, b1g, w2m, b2, w3m, b3)


def _tail_kernel(feat_ref, fcw_ref, fcb_ref, act_ref, emb_ref, rtg_ref,
                 rtgw_ref, rtgb_ref, wih_ref, whh_ref, bg_ref, head_ref,
                 logits_ref, hn_ref, cn_ref, h_scr, *, batch, n_steps,
                 hidden, n_actions):
    tb = batch * n_steps
    hd = hidden

    vis = jnp.tanh(
        jnp.dot(feat_ref[...], fcw_ref[...],
                preferred_element_type=jnp.float32) + fcb_ref[...])
    onehot = (jax.lax.broadcasted_iota(jnp.int32, (tb, n_actions), 1)
              == act_ref[...]).astype(jnp.float32)
    aemb = jnp.tanh(jnp.dot(onehot, emb_ref[...],
                            preferred_element_type=jnp.float32))
    remb = jnp.tanh(rtg_ref[...] * rtgw_ref[...] + rtgb_ref[...])
    zin = vis + aemb + remb

    gx = (jnp.dot(zin, wih_ref[...], preferred_element_type=jnp.float32)
          + bg_ref[...])
    whh = whh_ref[...]
    h = jnp.zeros((batch, hd), jnp.float32)
    c = jnp.zeros((batch, hd), jnp.float32)
    for t in range(n_steps):
        g = gx[t * batch:(t + 1) * batch, :] + jnp.dot(
            h, whh, preferred_element_type=jnp.float32)
        s = jax.nn.sigmoid(g[:, :3 * hd])
        gg = jnp.tanh(g[:, 3 * hd:])
        c = s[:, hd:2 * hd] * c + s[:, :hd] * gg
        h = s[:, 2 * hd:3 * hd] * jnp.tanh(c)
        h_scr[t * batch:(t + 1) * batch, :] = h
    logits_ref[...] = jnp.dot(h_scr[...], head_ref[...],
                              preferred_element_type=jnp.float32)
    hn_ref[...] = h
    cn_ref[...] = c


def _tail(feat_tm, fcw_t, fc_b, acts_tm, emb_table, rtg_tm, rtg_w, rtg_b,
          wih_t, whh_t, bg, head_pad, *, batch, n_steps, hidden, n_actions):
    tb = batch * n_steps
    e = fcw_t.shape[1]
    nh = head_pad.shape[1]

    def full(shape):
        return pl.BlockSpec(shape, lambda i: (0,) * len(shape))

    body = functools.partial(_tail_kernel, batch=batch, n_steps=n_steps,
                             hidden=hidden, n_actions=n_actions)
    return pl.pallas_call(
        body,
        out_shape=(
            jax.ShapeDtypeStruct((tb, nh), jnp.float32),
            jax.ShapeDtypeStruct((batch, hidden), jnp.float32),
            jax.ShapeDtypeStruct((batch, hidden), jnp.float32),
        ),
        grid_spec=pltpu.PrefetchScalarGridSpec(
            num_scalar_prefetch=0,
            grid=(1,),
            in_specs=[
                full((tb, 1024)), full((1024, e)), full((1, e)),
                full((tb, 1)), full(emb_table.shape), full((tb, 1)),
                full((1, e)), full((1, e)), full((e, 4 * hidden)),
                full((hidden, 4 * hidden)), full((1, 4 * hidden)),
                full((hidden, nh)),
            ],
            out_specs=(full((tb, nh)), full((batch, hidden)),
                       full((batch, hidden))),
            scratch_shapes=[pltpu.VMEM((tb, hidden), jnp.float32)],
        ),
        compiler_params=pltpu.CompilerParams(
            dimension_semantics=("arbitrary",)),
    )(feat_tm, fcw_t, fc_b.reshape(1, e), acts_tm, emb_table, rtg_tm,
      rtg_w.reshape(1, e), rtg_b.reshape(1, e), wih_t, whh_t,
      bg.reshape(1, 4 * hidden), head_pad)


def kernel(conv1_w, conv1_b, conv2_w, conv2_b, conv3_w, conv3_b, fc_w, fc_b,
           emb_table, rtg_w, rtg_b, w_ih, w_hh, b_ih, b_hh, head_w,
           frames, reward_to_go, previous_actions):
    b, f, img, _, _ = frames.shape
    e, hd, a = _E, _HD, _NA
    nb = 8 if b % 8 == 0 else (2 if b % 2 == 0 else 1)

    frames5 = jnp.transpose(frames.astype(jnp.float32), (0, 1, 4, 2, 3))

    eye4 = np.eye(4, dtype=np.float32)
    eye48 = np.eye(48, dtype=np.float32)
    m1 = jnp.asarray(np.repeat(np.repeat(eye48, 3, axis=0), 16, axis=1))
    w1c = jnp.transpose(conv1_w.astype(jnp.float32),
                        (2, 1, 3, 0))
    r3p = jnp.asarray(np.tile(np.eye(3, dtype=np.float32), (48, 1)))
    c16 = jnp.asarray(np.tile(np.eye(16, dtype=np.float32), (1, 48)))
    w1v = jnp.einsum('wi,aio,on->awn', r3p, w1c.reshape(9, 3, 16), c16)
    w1b = (w1v * m1[None]).reshape(3, 3, 144, 768)
    b1g = jnp.tile(conv1_b.astype(jnp.float32).reshape(1, 16), (1, 48))

    m2 = jnp.asarray(np.tile(
        np.repeat(np.repeat(eye4, 48, axis=0), 32, axis=1), (3, 1)))
    w2r = jnp.transpose(conv2_w.astype(jnp.float32),
                        (2, 3, 1, 0)).reshape(3, 48, 32)
    w2g = (jnp.tile(w2r, (1, 4, 4)).reshape(576, 128)) * m2
    b2g = jnp.tile(conv2_b.astype(jnp.float32).reshape(1, 32), (1, 4))

    m3 = jnp.asarray(np.tile(
        np.repeat(np.repeat(eye4, 128, axis=0), 64, axis=1), (4, 1)))
    w3r = jnp.transpose(conv3_w.astype(jnp.float32),
                        (2, 3, 1, 0)).reshape(4, 128, 64)
    w3g = (jnp.tile(w3r, (1, 4, 4)).reshape(2048, 256)) * m3
    b3g = jnp.tile(conv3_b.astype(jnp.float32).reshape(1, 64), (1, 4))

    feat_tm = _conv_tower(frames5, w1b, b1g, w2g, b2g, w3g, b3g,
                          b=b, f=f, nb=nb).reshape(f * b, 1024)

    fcw_t = jnp.transpose(fc_w.astype(jnp.float32).reshape(e, 64, 16),
                          (2, 1, 0)).reshape(1024, e)

    def gp(w):
        w = w.astype(jnp.float32)
        return jnp.concatenate(
            [w[:2 * hd], w[3 * hd:], w[2 * hd:3 * hd]], axis=0)

    wih_t = gp(w_ih).T
    whh_t = gp(w_hh).T
    bg = gp((b_ih + b_hh).astype(jnp.float32).reshape(4 * hd, 1)).reshape(-1)

    nh = -(-a // 128) * 128
    head_pad = jnp.pad(head_w.astype(jnp.float32).T, ((0, 0), (0, nh - a)))

    tb = b * f
    acts_tm = jnp.swapaxes(previous_actions, 0, 1).reshape(tb, 1)
    rtg_tm = jnp.swapaxes(reward_to_go.astype(jnp.float32),
                          0, 1).reshape(tb, 1)

    logits_pad, hn, cn = _tail(
        feat_tm, fcw_t, fc_b.astype(jnp.float32), acts_tm.astype(jnp.int32),
        emb_table.astype(jnp.float32), rtg_tm, rtg_w.astype(jnp.float32).T,
        rtg_b.astype(jnp.float32), wih_t, whh_t, bg, head_pad,
        batch=b, n_steps=f, hidden=hd, n_actions=a)

    logits = jnp.transpose(logits_pad.reshape(f, b, nh)[:, :, :a], (1, 0, 2))
    return logits, (hn[None], cn[None])


# speedup vs baseline: 23.9833x; 1.0295x over previous
import functools

import numpy as np
import jax
import jax.numpy as jnp
from jax.experimental import pallas as pl
from jax.experimental.pallas import tpu as pltpu

_E = 256
_HD = 128
_NA = 41


def _tower_kernel(x_ref, w1_ref, b1_ref, w2_ref, b2_ref, w3_ref, b3_ref,
                  o_ref):
    nb = x_ref.shape[0]
    m = 4 * nb

    def act(v, w, bias):
        return jnp.maximum(
            jnp.dot(v, w, preferred_element_type=jnp.float32) + bias, 0.0)

    acc = b1_ref[...]
    for r1 in range(3):
        for c in range(3):
            pieces = []
            for r3 in range(4):
                for r2 in range(3):
                    for oh3 in range(4):
                        h = 36 * oh3 + 9 * r3 + 3 * r2 + r1
                        pieces.append(x_ref[:, 0, c, h, :])
            xx = jnp.concatenate(pieces, axis=0)
            acc = acc + jnp.dot(xx, w1_ref[r1, c],
                                preferred_element_type=jnp.float32)
    y1 = jnp.maximum(acc, 0.0)

    y2p = []
    for g2 in range(4):
        p = jnp.concatenate(
            [jnp.concatenate(
                [y1[(r3 * 3 + r2) * m:(r3 * 3 + r2 + 1) * m,
                    g2 * 192:(g2 + 1) * 192] for r2 in range(3)], axis=1)
             for r3 in range(4)], axis=0)
        y2p.append(act(p, w2_ref[...], b2_ref[...]))
    y2 = jnp.concatenate(y2p, axis=1)

    p3 = jnp.concatenate([y2[r3 * m:(r3 + 1) * m, :]
                          for r3 in range(4)], axis=1)
    y3 = act(p3, w3_ref[...], b3_ref[...])

    for oh3 in range(4):
        o_ref[0, :, oh3 * 256:(oh3 + 1) * 256] = \
            y3[oh3 * nb:(oh3 + 1) * nb, :]


def _conv_tower(frames4, w1g, b1g, w2m, b2, w3m, b3, *, b, f, nb):
    jb = b // nb
    return pl.pallas_call(
        _tower_kernel,
        out_shape=jax.ShapeDtypeStruct((f * jb, nb, 1024), jnp.float32),
        grid_spec=pltpu.PrefetchScalarGridSpec(
            num_scalar_prefetch=0,
            grid=(jb, f),
            in_specs=[
                pl.BlockSpec((nb, 1, 3, 144, 144),
                             lambda j, t: (j, t, 0, 0, 0)),
                pl.BlockSpec((3, 3, 144, 768), lambda j, t: (0, 0, 0, 0)),
                pl.BlockSpec((1, 768), lambda j, t: (0, 0)),
                pl.BlockSpec((576, 128), lambda j, t: (0, 0)),
                pl.BlockSpec((1, 128), lambda j, t: (0, 0)),
                pl.BlockSpec((2048, 256), lambda j, t: (0, 0)),
                pl.BlockSpec((1, 256), lambda j, t: (0, 0)),
            ],
            out_specs=pl.BlockSpec((1, nb, 1024),
                                   lambda j, t: (t * jb + j, 0, 0)),
        ),
        compiler_params=pltpu.CompilerParams(
            dimension_semantics=("parallel", "parallel")),
    )(frames4, w1g, b1g, w2m, b2, w3m, b3)


def _tail_kernel(feat_ref, fcw_ref, fcb_ref, act_ref, emb_ref, rtg_ref,
                 rtgw_ref, rtgb_ref, wih_ref, whh_ref, bg_ref, head_ref,
                 logits_ref, hn_ref, cn_ref, h_scr, *, batch, n_steps,
                 hidden, n_actions):
    tb = batch * n_steps
    hd = hidden

    vis = jnp.tanh(
        jnp.dot(feat_ref[...], fcw_ref[...],
                preferred_element_type=jnp.float32) + fcb_ref[...])
    onehot = (jax.lax.broadcasted_iota(jnp.int32, (tb, n_actions), 1)
              == act_ref[...]).astype(jnp.float32)
    aemb = jnp.tanh(jnp.dot(onehot, emb_ref[...],
                            preferred_element_type=jnp.float32))
    remb = jnp.tanh(rtg_ref[...] * rtgw_ref[...] + rtgb_ref[...])
    zin = vis + aemb + remb

    gx = (jnp.dot(zin, wih_ref[...], preferred_element_type=jnp.float32)
          + bg_ref[...])
    whh = whh_ref[...]
    h = jnp.zeros((batch, hd), jnp.float32)
    c = jnp.zeros((batch, hd), jnp.float32)
    for t in range(n_steps):
        g = gx[t * batch:(t + 1) * batch, :] + jnp.dot(
            h, whh, preferred_element_type=jnp.float32)
        s = jax.nn.sigmoid(g[:, :3 * hd])
        gg = jnp.tanh(g[:, 3 * hd:])
        c = s[:, hd:2 * hd] * c + s[:, :hd] * gg
        h = s[:, 2 * hd:3 * hd] * jnp.tanh(c)
        h_scr[t * batch:(t + 1) * batch, :] = h
    logits_ref[...] = jnp.dot(h_scr[...], head_ref[...],
                              preferred_element_type=jnp.float32)
    hn_ref[...] = h
    cn_ref[...] = c


def _tail(feat_tm, fcw_t, fc_b, acts_tm, emb_table, rtg_tm, rtg_w, rtg_b,
          wih_t, whh_t, bg, head_pad, *, batch, n_steps, hidden, n_actions):
    tb = batch * n_steps
    e = fcw_t.shape[1]
    nh = head_pad.shape[1]

    def full(shape):
        return pl.BlockSpec(shape, lambda i: (0,) * len(shape))

    body = functools.partial(_tail_kernel, batch=batch, n_steps=n_steps,
                             hidden=hidden, n_actions=n_actions)
    return pl.pallas_call(
        body,
        out_shape=(
            jax.ShapeDtypeStruct((tb, nh), jnp.float32),
            jax.ShapeDtypeStruct((batch, hidden), jnp.float32),
            jax.ShapeDtypeStruct((batch, hidden), jnp.float32),
        ),
        grid_spec=pltpu.PrefetchScalarGridSpec(
            num_scalar_prefetch=0,
            grid=(1,),
            in_specs=[
                full((tb, 1024)), full((1024, e)), full((1, e)),
                full((tb, 1)), full(emb_table.shape), full((tb, 1)),
                full((1, e)), full((1, e)), full((e, 4 * hidden)),
                full((hidden, 4 * hidden)), full((1, 4 * hidden)),
                full((hidden, nh)),
            ],
            out_specs=(full((tb, nh)), full((batch, hidden)),
                       full((batch, hidden))),
            scratch_shapes=[pltpu.VMEM((tb, hidden), jnp.float32)],
        ),
        compiler_params=pltpu.CompilerParams(
            dimension_semantics=("arbitrary",)),
    )(feat_tm, fcw_t, fc_b.reshape(1, e), acts_tm, emb_table, rtg_tm,
      rtg_w.reshape(1, e), rtg_b.reshape(1, e), wih_t, whh_t,
      bg.reshape(1, 4 * hidden), head_pad)


def kernel(conv1_w, conv1_b, conv2_w, conv2_b, conv3_w, conv3_b, fc_w, fc_b,
           emb_table, rtg_w, rtg_b, w_ih, w_hh, b_ih, b_hh, head_w,
           frames, reward_to_go, previous_actions):
    b, f, img, _, _ = frames.shape
    e, hd, a = _E, _HD, _NA
    nb = 16 if b % 16 == 0 else (2 if b % 2 == 0 else 1)

    frames5 = jnp.transpose(frames.astype(jnp.float32), (0, 1, 4, 2, 3))

    eye4 = np.eye(4, dtype=np.float32)
    eye48 = np.eye(48, dtype=np.float32)
    m1 = jnp.asarray(np.repeat(np.repeat(eye48, 3, axis=0), 16, axis=1))
    w1c = jnp.transpose(conv1_w.astype(jnp.float32),
                        (2, 1, 3, 0))
    r3p = jnp.asarray(np.tile(np.eye(3, dtype=np.float32), (48, 1)))
    c16 = jnp.asarray(np.tile(np.eye(16, dtype=np.float32), (1, 48)))
    w1v = jnp.einsum('wi,aio,on->awn', r3p, w1c.reshape(9, 3, 16), c16)
    w1b = (w1v * m1[None]).reshape(3, 3, 144, 768)
    b1g = jnp.tile(conv1_b.astype(jnp.float32).reshape(1, 16), (1, 48))

    m2 = jnp.asarray(np.tile(
        np.repeat(np.repeat(eye4, 48, axis=0), 32, axis=1), (3, 1)))
    w2r = jnp.transpose(conv2_w.astype(jnp.float32),
                        (2, 3, 1, 0)).reshape(3, 48, 32)
    w2g = (jnp.tile(w2r, (1, 4, 4)).reshape(576, 128)) * m2
    b2g = jnp.tile(conv2_b.astype(jnp.float32).reshape(1, 32), (1, 4))

    m3 = jnp.asarray(np.tile(
        np.repeat(np.repeat(eye4, 128, axis=0), 64, axis=1), (4, 1)))
    w3r = jnp.transpose(conv3_w.astype(jnp.float32),
                        (2, 3, 1, 0)).reshape(4, 128, 64)
    w3g = (jnp.tile(w3r, (1, 4, 4)).reshape(2048, 256)) * m3
    b3g = jnp.tile(conv3_b.astype(jnp.float32).reshape(1, 64), (1, 4))

    feat_tm = _conv_tower(frames5, w1b, b1g, w2g, b2g, w3g, b3g,
                          b=b, f=f, nb=nb).reshape(f * b, 1024)

    fcw_t = jnp.transpose(fc_w.astype(jnp.float32).reshape(e, 64, 16),
                          (2, 1, 0)).reshape(1024, e)

    def gp(w):
        w = w.astype(jnp.float32)
        return jnp.concatenate(
            [w[:2 * hd], w[3 * hd:], w[2 * hd:3 * hd]], axis=0)

    wih_t = gp(w_ih).T
    whh_t = gp(w_hh).T
    bg = gp((b_ih + b_hh).astype(jnp.float32).reshape(4 * hd, 1)).reshape(-1)

    nh = -(-a // 128) * 128
    head_pad = jnp.pad(head_w.astype(jnp.float32).T, ((0, 0), (0, nh - a)))

    tb = b * f
    acts_tm = jnp.swapaxes(previous_actions, 0, 1).reshape(tb, 1)
    rtg_tm = jnp.swapaxes(reward_to_go.astype(jnp.float32),
                          0, 1).reshape(tb, 1)

    logits_pad, hn, cn = _tail(
        feat_tm, fcw_t, fc_b.astype(jnp.float32), acts_tm.astype(jnp.int32),
        emb_table.astype(jnp.float32), rtg_tm, rtg_w.astype(jnp.float32).T,
        rtg_b.astype(jnp.float32), wih_t, whh_t, bg, head_pad,
        batch=b, n_steps=f, hidden=hd, n_actions=a)

    logits = jnp.transpose(logits_pad.reshape(f, b, nh)[:, :, :a], (1, 0, 2))
    return logits, (hn[None], cn[None])


# speedup vs baseline: 24.6116x; 1.0262x over previous
import functools

import numpy as np
import jax
import jax.numpy as jnp
from jax.experimental import pallas as pl
from jax.experimental.pallas import tpu as pltpu

_E = 256
_HD = 128
_NA = 41


def _tower_kernel(x_ref, w1_ref, b1_ref, w2_ref, b2_ref, w3_ref, b3_ref,
                  o_ref):
    nb = x_ref.shape[0]
    m = 4 * nb

    def act(v, w, bias):
        return jnp.maximum(
            jnp.dot(v, w, preferred_element_type=jnp.float32) + bias, 0.0)

    acc = b1_ref[...]
    for r1 in range(3):
        for c in range(3):
            pieces = []
            for r3 in range(4):
                for r2 in range(3):
                    for oh3 in range(4):
                        h = 36 * oh3 + 9 * r3 + 3 * r2 + r1
                        pieces.append(x_ref[:, 0, c, h, :])
            xx = jnp.concatenate(pieces, axis=0)
            acc = acc + jnp.dot(xx, w1_ref[r1, c],
                                preferred_element_type=jnp.float32)
    y1 = jnp.maximum(acc, 0.0)

    y2p = []
    for g2 in range(4):
        p = jnp.concatenate(
            [jnp.concatenate(
                [y1[(r3 * 3 + r2) * m:(r3 * 3 + r2 + 1) * m,
                    g2 * 192:(g2 + 1) * 192] for r2 in range(3)], axis=1)
             for r3 in range(4)], axis=0)
        y2p.append(act(p, w2_ref[...], b2_ref[...]))
    y2 = jnp.concatenate(y2p, axis=1)

    p3 = jnp.concatenate([y2[r3 * m:(r3 + 1) * m, :]
                          for r3 in range(4)], axis=1)
    y3 = act(p3, w3_ref[...], b3_ref[...])

    for oh3 in range(4):
        o_ref[0, :, oh3 * 256:(oh3 + 1) * 256] = \
            y3[oh3 * nb:(oh3 + 1) * nb, :]


def _conv_tower(frames4, w1g, b1g, w2m, b2, w3m, b3, *, b, f, nb):
    jb = b // nb
    return pl.pallas_call(
        _tower_kernel,
        out_shape=jax.ShapeDtypeStruct((f * jb, nb, 1024), jnp.float32),
        grid_spec=pltpu.PrefetchScalarGridSpec(
            num_scalar_prefetch=0,
            grid=(jb, f),
            in_specs=[
                pl.BlockSpec((nb, 1, 3, 144, 144),
                             lambda j, t: (j, t, 0, 0, 0)),
                pl.BlockSpec((3, 3, 144, 768), lambda j, t: (0, 0, 0, 0)),
                pl.BlockSpec((1, 768), lambda j, t: (0, 0)),
                pl.BlockSpec((576, 128), lambda j, t: (0, 0)),
                pl.BlockSpec((1, 128), lambda j, t: (0, 0)),
                pl.BlockSpec((2048, 256), lambda j, t: (0, 0)),
                pl.BlockSpec((1, 256), lambda j, t: (0, 0)),
            ],
            out_specs=pl.BlockSpec((1, nb, 1024),
                                   lambda j, t: (t * jb + j, 0, 0)),
        ),
        compiler_params=pltpu.CompilerParams(
            dimension_semantics=("parallel", "parallel")),
    )(frames4, w1g, b1g, w2m, b2, w3m, b3)


def _tail_kernel(feat_ref, fcw_ref, fcb_ref, act_ref, emb_ref, rtg_ref,
                 rtgw_ref, rtgb_ref, wih_ref, whh_ref, bg_ref, head_ref,
                 logits_ref, hn_ref, cn_ref, h_scr, *, batch, n_steps,
                 hidden, n_actions):
    tb = batch * n_steps
    hd = hidden

    vis = jnp.tanh(
        jnp.dot(feat_ref[...], fcw_ref[...],
                preferred_element_type=jnp.float32) + fcb_ref[...])
    onehot = (jax.lax.broadcasted_iota(jnp.int32, (tb, n_actions), 1)
              == act_ref[...]).astype(jnp.float32)
    aemb = jnp.tanh(jnp.dot(onehot, emb_ref[...],
                            preferred_element_type=jnp.float32))
    remb = jnp.tanh(rtg_ref[...] * rtgw_ref[...] + rtgb_ref[...])
    zin = vis + aemb + remb

    tdot = functools.partial(
        jax.lax.dot_general, dimension_numbers=(((1,), (1,)), ((), ())),
        preferred_element_type=jnp.float32)
    gx = tdot(zin, wih_ref[...]) + bg_ref[...]
    whh = whh_ref[...]
    h = jnp.zeros((batch, hd), jnp.float32)
    c = jnp.zeros((batch, hd), jnp.float32)
    for t in range(n_steps):
        g = gx[t * batch:(t + 1) * batch, :] + tdot(h, whh)
        s = jax.nn.sigmoid(g[:, :2 * hd])
        o_g = jax.nn.sigmoid(g[:, 3 * hd:])
        gg = jnp.tanh(g[:, 2 * hd:3 * hd])
        c = s[:, hd:] * c + s[:, :hd] * gg
        h = o_g * jnp.tanh(c)
        h_scr[t * batch:(t + 1) * batch, :] = h
    logits_ref[...] = tdot(h_scr[...], head_ref[...])
    hn_ref[...] = h
    cn_ref[...] = c


def _tail(feat_tm, fcw_t, fc_b, acts_tm, emb_table, rtg_tm, rtg_w, rtg_b,
          w_ih, w_hh, bg, head_w, *, batch, n_steps, hidden, n_actions):
    tb = batch * n_steps
    e = fcw_t.shape[1]

    def full(shape):
        return pl.BlockSpec(shape, lambda i: (0,) * len(shape))

    body = functools.partial(_tail_kernel, batch=batch, n_steps=n_steps,
                             hidden=hidden, n_actions=n_actions)
    return pl.pallas_call(
        body,
        out_shape=(
            jax.ShapeDtypeStruct((tb, n_actions), jnp.float32),
            jax.ShapeDtypeStruct((batch, hidden), jnp.float32),
            jax.ShapeDtypeStruct((batch, hidden), jnp.float32),
        ),
        grid_spec=pltpu.PrefetchScalarGridSpec(
            num_scalar_prefetch=0,
            grid=(1,),
            in_specs=[
                full((tb, 1024)), full((1024, e)), full((1, e)),
                full((tb, 1)), full(emb_table.shape), full((tb, 1)),
                full((1, e)), full((1, e)), full((4 * hidden, e)),
                full((4 * hidden, hidden)), full((1, 4 * hidden)),
                full((n_actions, hidden)),
            ],
            out_specs=(full((tb, n_actions)), full((batch, hidden)),
                       full((batch, hidden))),
            scratch_shapes=[pltpu.VMEM((tb, hidden), jnp.float32)],
        ),
        compiler_params=pltpu.CompilerParams(
            dimension_semantics=("arbitrary",)),
    )(feat_tm, fcw_t, fc_b.reshape(1, e), acts_tm, emb_table, rtg_tm,
      rtg_w.reshape(1, e), rtg_b.reshape(1, e), w_ih, w_hh,
      bg.reshape(1, 4 * hidden), head_w)


def kernel(conv1_w, conv1_b, conv2_w, conv2_b, conv3_w, conv3_b, fc_w, fc_b,
           emb_table, rtg_w, rtg_b, w_ih, w_hh, b_ih, b_hh, head_w,
           frames, reward_to_go, previous_actions):
    b, f, img, _, _ = frames.shape
    e, hd, a = _E, _HD, _NA
    nb = 16 if b % 16 == 0 else (2 if b % 2 == 0 else 1)

    frames5 = jnp.transpose(frames.astype(jnp.float32), (0, 1, 4, 2, 3))

    eye4 = np.eye(4, dtype=np.float32)
    eye48 = np.eye(48, dtype=np.float32)
    m1 = jnp.asarray(np.repeat(np.repeat(eye48, 3, axis=0), 16, axis=1))
    w1c = jnp.transpose(conv1_w.astype(jnp.float32),
                        (2, 1, 3, 0))
    r3p = jnp.asarray(np.tile(np.eye(3, dtype=np.float32), (48, 1)))
    c16 = jnp.asarray(np.tile(np.eye(16, dtype=np.float32), (1, 48)))
    w1v = jnp.einsum('wi,aio,on->awn', r3p, w1c.reshape(9, 3, 16), c16)
    w1b = (w1v * m1[None]).reshape(3, 3, 144, 768)
    b1g = jnp.tile(conv1_b.astype(jnp.float32).reshape(1, 16), (1, 48))

    m2 = jnp.asarray(np.tile(
        np.repeat(np.repeat(eye4, 48, axis=0), 32, axis=1), (3, 1)))
    w2r = jnp.transpose(conv2_w.astype(jnp.float32),
                        (2, 3, 1, 0)).reshape(3, 48, 32)
    w2g = (jnp.tile(w2r, (1, 4, 4)).reshape(576, 128)) * m2
    b2g = jnp.tile(conv2_b.astype(jnp.float32).reshape(1, 32), (1, 4))

    m3 = jnp.asarray(np.tile(
        np.repeat(np.repeat(eye4, 128, axis=0), 64, axis=1), (4, 1)))
    w3r = jnp.transpose(conv3_w.astype(jnp.float32),
                        (2, 3, 1, 0)).reshape(4, 128, 64)
    w3g = (jnp.tile(w3r, (1, 4, 4)).reshape(2048, 256)) * m3
    b3g = jnp.tile(conv3_b.astype(jnp.float32).reshape(1, 64), (1, 4))

    feat_tm = _conv_tower(frames5, w1b, b1g, w2g, b2g, w3g, b3g,
                          b=b, f=f, nb=nb).reshape(f * b, 1024)

    fcw_t = jnp.transpose(fc_w.astype(jnp.float32).reshape(e, 64, 16),
                          (2, 1, 0)).reshape(1024, e)

    bg = (b_ih + b_hh).astype(jnp.float32)

    tb = b * f
    acts_tm = jnp.swapaxes(previous_actions, 0, 1).reshape(tb, 1)
    rtg_tm = jnp.swapaxes(reward_to_go.astype(jnp.float32),
                          0, 1).reshape(tb, 1)

    logits_tm, hn, cn = _tail(
        feat_tm, fcw_t, fc_b.astype(jnp.float32), acts_tm.astype(jnp.int32),
        emb_table.astype(jnp.float32), rtg_tm, rtg_w.astype(jnp.float32).T,
        rtg_b.astype(jnp.float32), w_ih.astype(jnp.float32),
        w_hh.astype(jnp.float32), bg, head_w.astype(jnp.float32),
        batch=b, n_steps=f, hidden=hd, n_actions=a)

    logits = jnp.transpose(logits_tm.reshape(f, b, a), (1, 0, 2))
    return logits, (hn[None], cn[None])
